# baseline (speedup 1.0000x reference)
import functools

import jax
import jax.numpy as jnp
from jax.experimental import pallas as pl
from jax.experimental.pallas import tpu as pltpu

N = 4096
DIN = 128
DH = 64
DOUT = 128
H = 4

LOG2E = 1.4426950408889634
NEG = -9e15 * LOG2E


def _proj_body(x_ref, w_ref, a_ref, h_ref, f_ref, ft_ref):
    x = x_ref[...]
    hs = [
        jnp.dot(x, w_ref[hh], preferred_element_type=jnp.float32)
        for hh in range(H)
    ]
    h_ref[...] = jnp.concatenate(hs, axis=1)
    cols = []
    for hh in range(H):
        cols.append(jnp.sum(hs[hh] * a_ref[hh : hh + 1, :DH], axis=1, keepdims=True))
    for hh in range(H):
        cols.append(jnp.sum(hs[hh] * a_ref[hh : hh + 1, DH:], axis=1, keepdims=True))
    f = jnp.concatenate(cols, axis=1)
    f_ref[...] = f
    ft_ref[...] = f.T


def _proj(x, w1, a1s, bi):
    return pl.pallas_call(
        _proj_body,
        grid=(N // bi,),
        in_specs=[
            pl.BlockSpec((bi, DIN), lambda i: (i, 0)),
            pl.BlockSpec((H, DIN, DH), lambda i: (0, 0, 0)),
            pl.BlockSpec((8, 2 * DH), lambda i: (0, 0)),
        ],
        out_specs=[
            pl.BlockSpec((bi, H * DH), lambda i: (i, 0)),
            pl.BlockSpec((bi, 8), lambda i: (i, 0)),
            pl.BlockSpec((8, bi), lambda i: (0, i)),
        ],
        out_shape=[
            jax.ShapeDtypeStruct((N, H * DH), jnp.float32),
            jax.ShapeDtypeStruct((N, 8), jnp.float32),
            jax.ShapeDtypeStruct((8, N), jnp.float32),
        ],
        compiler_params=pltpu.CompilerParams(
            dimension_semantics=("parallel",),
        ),
    )(x, w1, a1s)


def _flash_body(nheads, dh, adj_ref, a_ref, h_ref, fb_ref, ft_ref, *rest):
    adj = adj_ref[...]
    ab = a_ref[...]
    outs = []
    for hh in range(nheads):
        f1c = fb_ref[:, hh : hh + 1]
        f2r = ft_ref[nheads + hh : nheads + hh + 1, :]
        e = f1c + f2r
        e = jnp.maximum(e, 0.2 * e)
        e = jnp.where(adj > 0.0, e, NEG)
        m = jnp.max(e, axis=1, keepdims=True)
        p = jnp.exp2(e - m)
        z = jnp.sum(p, axis=1, keepdims=True)
        o = jnp.dot(
            p * ab,
            h_ref[:, hh * dh : (hh + 1) * dh],
            preferred_element_type=jnp.float32,
        )
        o = o / z
        o = jnp.where(o > 0.0, o, jnp.exp(jnp.minimum(o, 0.0)) - 1.0)
        outs.append(o)
    xm = outs[0] if nheads == 1 else jnp.concatenate(outs, axis=1)
    if rest and len(rest) == 5:
        w2_ref, a2_ref, h2_ref, g_ref, gt_ref = rest
        h2 = jnp.dot(xm, w2_ref[...], preferred_element_type=jnp.float32)
        h2_ref[...] = h2
        g1 = jnp.sum(h2 * a2_ref[0:1, :], axis=1, keepdims=True)
        g2 = jnp.sum(h2 * a2_ref[1:2, :], axis=1, keepdims=True)
        g = jnp.concatenate(
            [g1, g2, jnp.zeros((g1.shape[0], 6), jnp.float32)], axis=1
        )
        g_ref[...] = g
        gt_ref[...] = g.T
    else:
        rest[0][...] = xm


def _flash(adj, a, h, fb, ft, nheads, dh, bi, w2=None, a2s=None):
    dh_tot = nheads * dh
    in_specs = [
        pl.BlockSpec((bi, N), lambda i: (i, 0)),
        pl.BlockSpec((bi, N), lambda i: (i, 0)),
        pl.BlockSpec((N, dh_tot), lambda i: (0, 0)),
        pl.BlockSpec((bi, 8), lambda i: (i, 0)),
        pl.BlockSpec((8, N), lambda i: (0, 0)),
    ]
    args = [adj, a, h, fb, ft]
    if w2 is not None:
        dout = w2.shape[1]
        in_specs += [
            pl.BlockSpec((dh_tot, dout), lambda i: (0, 0)),
            pl.BlockSpec((8, dout), lambda i: (0, 0)),
        ]
        args += [w2, a2s]
        out_specs = [
            pl.BlockSpec((bi, dout), lambda i: (i, 0)),
            pl.BlockSpec((bi, 8), lambda i: (i, 0)),
            pl.BlockSpec((8, bi), lambda i: (0, i)),
        ]
        out_shape = [
            jax.ShapeDtypeStruct((N, dout), jnp.float32),
            jax.ShapeDtypeStruct((N, 8), jnp.float32),
            jax.ShapeDtypeStruct((8, N), jnp.float32),
        ]
    else:
        out_specs = [pl.BlockSpec((bi, dh_tot), lambda i: (i, 0))]
        out_shape = [jax.ShapeDtypeStruct((N, dh_tot), jnp.float32)]

    body = functools.partial(_flash_body, nheads, dh)
    return pl.pallas_call(
        body,
        grid=(N // bi,),
        in_specs=in_specs,
        out_specs=out_specs,
        out_shape=out_shape,
        compiler_params=pltpu.CompilerParams(
            dimension_semantics=("parallel",),
        ),
    )(*args)


@jax.jit
def kernel(x, adj, A, W1, a1, W2, a2):
    a1s = jnp.pad(a1 * LOG2E, ((0, 8 - H), (0, 0)))
    a2s = jnp.pad(a2.reshape(2, DOUT) * LOG2E, ((0, 6), (0, 0)))

    h1, f1, f1t = _proj(x, W1, a1s, 512)
    h2, g, gt = _flash(adj, A, h1, f1, f1t, H, DH, 256, w2=W2, a2s=a2s)
    out = _flash(adj, A, h2, g, gt, 1, DOUT, 256)
    return out[0]


# speedup vs baseline: 1.0326x; 1.0326x over previous
import jax
import jax.numpy as jnp
from jax.experimental import pallas as pl
from jax.experimental.pallas import tpu as pltpu

N = 4096
DIN = 128
DH = 64
DOUT = 128
H = 4

BP = 512
BI = 256
NP = N // BP
NB = N // BI

LOG2E = 1.4426950408889634
NEG = -9e15 * LOG2E


def _attend(adj, ab, f1c, f2r, hv):
    e = f1c + f2r
    e = jnp.maximum(e, 0.2 * e)
    e = jnp.where(adj > 0.0, e, NEG)
    m = jnp.max(e, axis=1, keepdims=True)
    p = jnp.exp2(e - m)
    z = jnp.sum(p, axis=1, keepdims=True)
    o = jnp.dot(p * ab, hv, preferred_element_type=jnp.float32)
    o = o / z
    return jnp.where(o > 0.0, o, jnp.exp(jnp.minimum(o, 0.0)) - 1.0)


def _body(x_ref, w1_ref, a1_ref, adj_ref, a_ref, w2_ref, a2_ref, out_ref,
          h1_s, f1_s, f1t_s, h2_s, g_s, gt_s):
    i = pl.program_id(0)

    @pl.when(i < NP)
    def _proj():
        x = x_ref[...]
        hs = [
            jnp.dot(x, w1_ref[hh], preferred_element_type=jnp.float32)
            for hh in range(H)
        ]
        r = pl.ds(pl.multiple_of(i * BP, BP), BP)
        h1_s[r, :] = jnp.concatenate(hs, axis=1)
        cols = []
        for hh in range(H):
            cols.append(
                jnp.sum(hs[hh] * a1_ref[hh : hh + 1, :DH], axis=1, keepdims=True)
            )
        for hh in range(H):
            cols.append(
                jnp.sum(hs[hh] * a1_ref[hh : hh + 1, DH:], axis=1, keepdims=True)
            )
        f = jnp.concatenate(cols, axis=1)
        f1_s[r, :] = f
        f1t_s[:, r] = f.T

    @pl.when((i >= NP) & (i < NP + NB))
    def _layer1():
        r = pl.ds(pl.multiple_of((i - NP) * BI, BI), BI)
        adj = adj_ref[...]
        ab = a_ref[...]
        outs = []
        for hh in range(H):
            outs.append(
                _attend(
                    adj,
                    ab,
                    f1_s[r, hh : hh + 1],
                    f1t_s[H + hh : H + hh + 1, :],
                    h1_s[:, hh * DH : (hh + 1) * DH],
                )
            )
        xm = jnp.concatenate(outs, axis=1)
        h2 = jnp.dot(xm, w2_ref[...], preferred_element_type=jnp.float32)
        h2_s[r, :] = h2
        g1 = jnp.sum(h2 * a2_ref[0:1, :], axis=1, keepdims=True)
        g2 = jnp.sum(h2 * a2_ref[1:2, :], axis=1, keepdims=True)
        g_s[r, 0:1] = g1
        gt_s[1:2, r] = g2.T

    @pl.when(i >= NP + NB)
    def _layer2():
        r = pl.ds(pl.multiple_of((i - NP - NB) * BI, BI), BI)
        out_ref[...] = _attend(
            adj_ref[...],
            a_ref[...],
            g_s[r, 0:1],
            gt_s[1:2, :],
            h2_s[...],
        )


@jax.jit
def kernel(x, adj, A, W1, a1, W2, a2):
    a1s = jnp.pad(a1 * LOG2E, ((0, 8 - H), (0, 0)))
    a2s = jnp.pad(a2.reshape(2, DOUT) * LOG2E, ((0, 6), (0, 0)))

    def flash_map(i):
        return (jnp.where(i < NP, 0, (i - NP) % NB), 0)

    out = pl.pallas_call(
        _body,
        grid=(NP + 2 * NB,),
        in_specs=[
            pl.BlockSpec((BP, DIN), lambda i: (jnp.minimum(i, NP - 1), 0)),
            pl.BlockSpec((H, DIN, DH), lambda i: (0, 0, 0)),
            pl.BlockSpec((8, 2 * DH), lambda i: (0, 0)),
            pl.BlockSpec((BI, N), flash_map),
            pl.BlockSpec((BI, N), flash_map),
            pl.BlockSpec((H * DH, DOUT), lambda i: (0, 0)),
            pl.BlockSpec((8, DOUT), lambda i: (0, 0)),
        ],
        out_specs=pl.BlockSpec(
            (BI, DOUT), lambda i: (jnp.maximum(i - NP - NB, 0), 0)
        ),
        out_shape=jax.ShapeDtypeStruct((N, DOUT), jnp.float32),
        scratch_shapes=[
            pltpu.VMEM((N, H * DH), jnp.float32),
            pltpu.VMEM((N, 8), jnp.float32),
            pltpu.VMEM((8, N), jnp.float32),
            pltpu.VMEM((N, DOUT), jnp.float32),
            pltpu.VMEM((N, 8), jnp.float32),
            pltpu.VMEM((8, N), jnp.float32),
        ],
        compiler_params=pltpu.CompilerParams(
            dimension_semantics=("arbitrary",),
        ),
    )(x, W1, a1s, adj, A, W2, a2s)
    return out
